# Initial kernel scaffold; baseline (speedup 1.0000x reference)
import functools

import jax
import jax.numpy as jnp
from jax import lax
from jax.experimental import pallas as pl
from jax.experimental.pallas import tpu as pltpu
from jax.experimental.pallas import tpu_sc as plsc

N = 100000
C = 128
G = 64
NC = 2
NS = 16
NW = NC * NS
RPW = N // NW
K = 125
CH = RPW // K
L = 16
CJ = C // L


def _body(feat, batch3, wflat, b16, out, acc, idx_v, fbuf, wv, bv, z4):
    cid = lax.axis_index("c")
    sid = lax.axis_index("s")
    wid = cid * NS + sid

    pltpu.sync_copy(wflat, wv)
    pltpu.sync_copy(b16, bv)
    pltpu.sync_copy(batch3.at[wid], idx_v)

    zero = jnp.zeros((L,), jnp.float32)
    for r in range(G // NS):
        for j in range(CJ):
            z4[r, pl.ds(L * j, L)] = zero
    pltpu.sync_copy(z4, acc.at[pl.ds(sid * (G // NS), G // NS)])
    plsc.subcore_barrier()

    bvec = bv[...]
    wregs = [wv[pl.ds(L * j, L)] for j in range(CJ)]

    def chunk_body(c, carry):
        row0 = wid * RPW + c * K
        pltpu.sync_copy(feat.at[pl.ds(row0, K)], fbuf)

        def row_body(r, rc):
            fr = [fbuf[r, pl.ds(L * j, L)] for j in range(CJ)]
            p = fr[0] * wregs[0]
            for j in range(1, CJ):
                p = p + fr[j] * wregs[j]
            s = jnp.sum(p)
            t = bvec + s
            wgt = 1.0 / (1.0 + jnp.exp(-t))
            for j in range(CJ):
                fbuf[r, pl.ds(L * j, L)] = fr[j] * wgt
            return rc

        lax.fori_loop(0, K, row_body, 0)
        pltpu.sync_copy(fbuf, acc.at[idx_v.at[c]], add=True)
        return carry

    lax.fori_loop(0, CH, chunk_body, 0)
    plsc.subcore_barrier()

    @pl.when(sid == 0)
    def _():
        pltpu.sync_copy(acc, out.at[cid])


@jax.jit
def _pooling(features, batch3, wflat, b16):
    mesh = plsc.VectorSubcoreMesh(core_axis_name="c", subcore_axis_name="s")
    kfn = functools.partial(
        pl.kernel,
        mesh=mesh,
        out_type=jax.ShapeDtypeStruct((NC, G, C), jnp.float32),
        scratch_types=[
            pltpu.VMEM_SHARED((G, C), jnp.float32),
            pltpu.VMEM((CH, K), jnp.int32),
            pltpu.VMEM((K, C), jnp.float32),
            pltpu.VMEM((C,), jnp.float32),
            pltpu.VMEM((L,), jnp.float32),
            pltpu.VMEM((G // NS, C), jnp.float32),
        ],
    )(_body)
    return kfn(features, batch3, wflat, b16)


def kernel(features, batch, W, b):
    batch3 = batch.astype(jnp.int32).reshape(NW, CH, K)
    wflat = W.reshape(C).astype(jnp.float32)
    b16 = jnp.broadcast_to(b.reshape(()).astype(jnp.float32), (L,))
    partials = _pooling(features, batch3, wflat, b16)
    return partials[0] + partials[1]


# speedup vs baseline: 2.0755x; 2.0755x over previous
import functools

import jax
import jax.numpy as jnp
from jax import lax
from jax.experimental import pallas as pl
from jax.experimental.pallas import tpu as pltpu
from jax.experimental.pallas import tpu_sc as plsc

N = 100000
C = 128
G = 64
NC = 2
NS = 16
NW = NC * NS
K = 160
KH = K // 2
TCH = N // K
FULL = TCH % NW
CPW = TCH // NW
L = 16
CJ = C // L


def _body(feat, batch3, wflat, b16, out, acc, idx_v, fbuf, wv, bv, z8):
    cid = lax.axis_index("c")
    sid = lax.axis_index("s")
    wid = cid * NS + sid

    pltpu.sync_copy(wflat, wv)
    pltpu.sync_copy(b16, bv)

    zero = jnp.zeros((L,), jnp.float32)
    for r in range(8):
        for j in range(CJ):
            z8[r, pl.ds(L * j, L)] = zero

    @pl.when(sid < 8)
    def _():
        pltpu.sync_copy(z8, acc.at[pl.ds(sid * 8, 8)])

    plsc.subcore_barrier()

    bvec = bv[...]
    wregs = [wv[pl.ds(L * j, L)] for j in range(CJ)]
    nch = jnp.where(wid < FULL, CPW + 1, CPW)

    lanes = lax.iota(jnp.int32, L)
    bfly = [lanes ^ sh for sh in (8, 4, 2, 1)]

    dnums = lax.GatherDimensionNumbers(
        offset_dims=(), collapsed_slice_dims=(0,), start_index_map=(0,)
    )

    def take16(x, idx):
        return lax.gather(
            x,
            idx[:, None],
            dnums,
            slice_sizes=(1,),
            mode=lax.GatherScatterMode.PROMISE_IN_BOUNDS,
        )

    def lane_sum_splat(x):
        for idx in bfly:
            x = x + take16(x, idx)
        return x

    def chunk_body(t, carry):
        c = wid + t * NW
        pltpu.sync_copy(feat.at[pl.ds(c * K, K)], fbuf)
        pltpu.sync_copy(batch3.at[c], idx_v)

        def row_body(r, rc):
            fr = [fbuf[r, pl.ds(L * j, L)] for j in range(CJ)]
            p = fr[0] * wregs[0]
            for j in range(1, CJ):
                p = p + fr[j] * wregs[j]
            s = lane_sum_splat(p)
            t_ = bvec + s
            wgt = 1.0 / (1.0 + jnp.exp(-t_))
            for j in range(CJ):
                fbuf[r, pl.ds(L * j, L)] = fr[j] * wgt
            return rc

        lax.fori_loop(0, K, row_body, 0)
        for h in range(2):
            pltpu.sync_copy(
                fbuf.at[pl.ds(h * KH, KH)], acc.at[idx_v.at[h]], add=True
            )
        return carry

    lax.fori_loop(0, nch, chunk_body, 0)
    plsc.subcore_barrier()

    @pl.when(sid == 0)
    def _():
        pltpu.sync_copy(acc, out.at[cid])


@jax.jit
def _pooling(features, batch3, wflat, b16):
    mesh = plsc.VectorSubcoreMesh(core_axis_name="c", subcore_axis_name="s")
    kfn = functools.partial(
        pl.kernel,
        mesh=mesh,
        out_type=jax.ShapeDtypeStruct((NC, G, C), jnp.float32),
        scratch_types=[
            pltpu.VMEM_SHARED((G, C), jnp.float32),
            pltpu.VMEM((2, KH), jnp.int32),
            pltpu.VMEM((K, C), jnp.float32),
            pltpu.VMEM((C,), jnp.float32),
            pltpu.VMEM((L,), jnp.float32),
            pltpu.VMEM((8, C), jnp.float32),
        ],
    )(_body)
    return kfn(features, batch3, wflat, b16)


def kernel(features, batch, W, b):
    batch3 = batch.astype(jnp.int32).reshape(TCH, 2, KH)
    wflat = W.reshape(C).astype(jnp.float32)
    b16 = jnp.broadcast_to(b.reshape(()).astype(jnp.float32), (L,))
    partials = _pooling(features, batch3, wflat, b16)
    return partials[0] + partials[1]


# speedup vs baseline: 3.1633x; 1.5241x over previous
import functools

import jax
import jax.numpy as jnp
from jax import lax
from jax.experimental import pallas as pl
from jax.experimental.pallas import tpu as pltpu
from jax.experimental.pallas import tpu_sc as plsc

N = 100000
C = 128
G = 64
NC = 2
NS = 16
NW = NC * NS
K = 160
KH = K // 2
TCH = N // K
FULL = TCH % NW
CPW = TCH // NW
L = 16
CJ = C // L


def _body(feat, batch3, wflat, b16, out, acc, idx_v, fbuf, wv, bv, z8):
    cid = lax.axis_index("c")
    sid = lax.axis_index("s")
    wid = cid * NS + sid

    pltpu.sync_copy(wflat, wv)
    pltpu.sync_copy(b16, bv)

    zero = jnp.zeros((L,), jnp.float32)
    for r in range(8):
        for j in range(CJ):
            z8[r, pl.ds(L * j, L)] = zero

    @pl.when(sid < 8)
    def _():
        pltpu.sync_copy(z8, acc.at[pl.ds(sid * 8, 8)])

    plsc.subcore_barrier()

    bvec = bv[...]
    wregs = [wv[pl.ds(L * j, L)] for j in range(CJ)]
    nch = jnp.where(wid < FULL, CPW + 1, CPW)

    lanes = lax.iota(jnp.int32, L)
    bfly = [lanes ^ sh for sh in (8, 4, 2, 1)]

    dnums = lax.GatherDimensionNumbers(
        offset_dims=(), collapsed_slice_dims=(0,), start_index_map=(0,)
    )

    def take16(x, idx):
        return lax.gather(
            x,
            idx[:, None],
            dnums,
            slice_sizes=(1,),
            mode=lax.GatherScatterMode.PROMISE_IN_BOUNDS,
        )

    def lane_sum_splat(x):
        for idx in bfly:
            x = x + take16(x, idx)
        return x

    U = 4

    def scale_row(r):
        fr = [fbuf[r, pl.ds(L * j, L)] for j in range(CJ)]
        m = [fr[j] * wregs[j] for j in range(CJ)]
        while len(m) > 1:
            m = [m[2 * i] + m[2 * i + 1] for i in range(len(m) // 2)]
        s = lane_sum_splat(m[0])
        t_ = bvec + s
        wgt = 1.0 / (1.0 + jnp.exp(-t_))
        for j in range(CJ):
            fbuf[r, pl.ds(L * j, L)] = fr[j] * wgt

    def chunk_body(t, carry):
        c = wid + t * NW
        pltpu.sync_copy(feat.at[pl.ds(c * K, K)], fbuf)
        pltpu.sync_copy(batch3.at[c], idx_v)

        def row_body(rq, rc):
            for i in range(U):
                scale_row(rq * U + i)
            return rc

        lax.fori_loop(0, K // U, row_body, 0)
        for h in range(2):
            pltpu.sync_copy(
                fbuf.at[pl.ds(h * KH, KH)], acc.at[idx_v.at[h]], add=True
            )
        return carry

    lax.fori_loop(0, nch, chunk_body, 0)
    plsc.subcore_barrier()

    @pl.when(sid == 0)
    def _():
        pltpu.sync_copy(acc, out.at[cid])


@jax.jit
def _pooling(features, batch3, wflat, b16):
    mesh = plsc.VectorSubcoreMesh(core_axis_name="c", subcore_axis_name="s")
    kfn = functools.partial(
        pl.kernel,
        mesh=mesh,
        out_type=jax.ShapeDtypeStruct((NC, G, C), jnp.float32),
        scratch_types=[
            pltpu.VMEM_SHARED((G, C), jnp.float32),
            pltpu.VMEM((2, KH), jnp.int32),
            pltpu.VMEM((K, C), jnp.float32),
            pltpu.VMEM((C,), jnp.float32),
            pltpu.VMEM((L,), jnp.float32),
            pltpu.VMEM((8, C), jnp.float32),
        ],
    )(_body)
    return kfn(features, batch3, wflat, b16)


def kernel(features, batch, W, b):
    batch3 = batch.astype(jnp.int32).reshape(TCH, 2, KH)
    wflat = W.reshape(C).astype(jnp.float32)
    b16 = jnp.broadcast_to(b.reshape(()).astype(jnp.float32), (L,))
    partials = _pooling(features, batch3, wflat, b16)
    return partials[0] + partials[1]


# speedup vs baseline: 4.3666x; 1.3804x over previous
import functools

import jax
import jax.numpy as jnp
from jax import lax
from jax.experimental import pallas as pl
from jax.experimental.pallas import tpu as pltpu
from jax.experimental.pallas import tpu_sc as plsc

N = 100000
C = 128
G = 64
NC = 2
NS = 16
NW = NC * NS
K = 160
KH = K // 2
TCH = N // K
FULL = TCH % NW
CPW = TCH // NW
L = 16
CJ = C // L
U = 4


def _body(feat, batch3, wflat, b16, out, acc, idx_v, fbuf, wv, bv, z8,
          semf, semi):
    cid = lax.axis_index("c")
    sid = lax.axis_index("s")
    wid = cid * NS + sid

    pltpu.sync_copy(wflat, wv)
    pltpu.sync_copy(b16, bv)

    zero = jnp.zeros((L,), jnp.float32)
    for r in range(8):
        for j in range(CJ):
            z8[r, pl.ds(L * j, L)] = zero

    @pl.when(sid < 8)
    def _():
        pltpu.sync_copy(z8, acc.at[pl.ds(sid * 8, 8)])

    plsc.subcore_barrier()

    bvec = bv[...]
    wregs = [wv[pl.ds(L * j, L)] for j in range(CJ)]
    nch = jnp.where(wid < FULL, CPW + 1, CPW)
    start = wid * CPW + jnp.minimum(wid, FULL)

    lanes = lax.iota(jnp.int32, L)
    bfly = [lanes ^ sh for sh in (8, 4, 2, 1)]
    dnums = lax.GatherDimensionNumbers(
        offset_dims=(), collapsed_slice_dims=(0,), start_index_map=(0,)
    )

    def take16(x, idx):
        return lax.gather(
            x,
            idx[:, None],
            dnums,
            slice_sizes=(1,),
            mode=lax.GatherScatterMode.PROMISE_IN_BOUNDS,
        )

    def lane_sum_splat(x):
        for idx in bfly:
            x = x + take16(x, idx)
        return x

    def issue_fetch(c, p):
        pltpu.async_copy(feat.at[pl.ds(c * K, K)], fbuf.at[p], semf.at[p])
        pltpu.async_copy(batch3.at[c], idx_v.at[p], semi.at[p])

    def wait_fetch(p):
        pltpu.make_async_copy(feat.at[pl.ds(0, K)], fbuf.at[p], semf.at[p]).wait()
        pltpu.make_async_copy(batch3.at[0], idx_v.at[p], semi.at[p]).wait()

    def process(c, t, p):
        wait_fetch(p)
        fb = fbuf.at[p]

        def scale_row(r):
            fr = [fb[r, pl.ds(L * j, L)] for j in range(CJ)]
            m = [fr[j] * wregs[j] for j in range(CJ)]
            while len(m) > 1:
                m = [m[2 * i] + m[2 * i + 1] for i in range(len(m) // 2)]
            s = lane_sum_splat(m[0])
            t_ = bvec + s
            wgt = 1.0 / (1.0 + jnp.exp(-t_))
            for j in range(CJ):
                fb[r, pl.ds(L * j, L)] = fr[j] * wgt

        def row_body(rq, rc):
            for i in range(U):
                scale_row(rq * U + i)
            return rc

        lax.fori_loop(0, K // U, row_body, 0)

        for h in range(2):
            pltpu.sync_copy(
                fb.at[pl.ds(h * KH, KH)], acc.at[idx_v.at[p, h]], add=True
            )

        @pl.when(t + 2 < nch)
        def _():
            issue_fetch(c + 2, p)

    issue_fetch(start, 0)
    issue_fetch(start + 1, 1)

    def chunk_body(t, carry):
        c = start + t

        @pl.when(t % 2 == 0)
        def _():
            process(c, t, 0)

        @pl.when(t % 2 == 1)
        def _():
            process(c, t, 1)

        return carry

    lax.fori_loop(0, nch, chunk_body, 0)
    plsc.subcore_barrier()

    @pl.when(sid == 0)
    def _():
        pltpu.sync_copy(acc, out.at[cid])


@jax.jit
def _pooling(features, batch3, wflat, b16):
    mesh = plsc.VectorSubcoreMesh(core_axis_name="c", subcore_axis_name="s")
    kfn = functools.partial(
        pl.kernel,
        mesh=mesh,
        out_type=jax.ShapeDtypeStruct((NC, G, C), jnp.float32),
        scratch_types=[
            pltpu.VMEM_SHARED((G, C), jnp.float32),
            pltpu.VMEM((2, 2, KH), jnp.int32),
            pltpu.VMEM((2, K, C), jnp.float32),
            pltpu.VMEM((C,), jnp.float32),
            pltpu.VMEM((L,), jnp.float32),
            pltpu.VMEM((8, C), jnp.float32),
            pltpu.SemaphoreType.DMA((2,)),
            pltpu.SemaphoreType.DMA((2,)),
        ],
    )(_body)
    return kfn(features, batch3, wflat, b16)


def kernel(features, batch, W, b):
    batch3 = batch.astype(jnp.int32).reshape(TCH, 2, KH)
    wflat = W.reshape(C).astype(jnp.float32)
    b16 = jnp.broadcast_to(b.reshape(()).astype(jnp.float32), (L,))
    partials = _pooling(features, batch3, wflat, b16)
    return partials[0] + partials[1]


# speedup vs baseline: 6.0080x; 1.3759x over previous
import functools

import jax
import jax.numpy as jnp
from jax import lax
from jax.experimental import pallas as pl
from jax.experimental.pallas import tpu as pltpu
from jax.experimental.pallas import tpu_sc as plsc

N = 100000
C = 128
G = 64
NC = 2
NS = 16
NW = NC * NS
K = 160
KH = K // 2
TCH = N // K
FULL = TCH % NW
CPW = TCH // NW
L = 16
CJ = C // L
U = 4


def _body(feat, batch3, wflat, b16, out, acc, idx_v, idx_s, fbuf, sbuf,
          wv, bv, z8, semf, semi, sems):
    cid = lax.axis_index("c")
    sid = lax.axis_index("s")
    wid = cid * NS + sid

    pltpu.sync_copy(wflat, wv)
    pltpu.sync_copy(b16, bv)

    zero = jnp.zeros((L,), jnp.float32)
    for r in range(8):
        for j in range(CJ):
            z8[r, pl.ds(L * j, L)] = zero

    @pl.when(sid < 8)
    def _():
        pltpu.sync_copy(z8, acc.at[pl.ds(sid * 8, 8)])

    plsc.subcore_barrier()

    bvec = bv[...]
    wregs = [wv[pl.ds(L * j, L)] for j in range(CJ)]
    nch = jnp.where(wid < FULL, CPW + 1, CPW)
    start = wid * CPW + jnp.minimum(wid, FULL)

    lanes = lax.iota(jnp.int32, L)
    bfly = [lanes ^ sh for sh in (8, 4, 2, 1)]
    dnums = lax.GatherDimensionNumbers(
        offset_dims=(), collapsed_slice_dims=(0,), start_index_map=(0,)
    )

    def take16(x, idx):
        return lax.gather(
            x,
            idx[:, None],
            dnums,
            slice_sizes=(1,),
            mode=lax.GatherScatterMode.PROMISE_IN_BOUNDS,
        )

    def lane_sum_splat(x):
        for idx in bfly:
            x = x + take16(x, idx)
        return x

    def issue_fetch(c, p):
        pltpu.async_copy(feat.at[pl.ds(c * K, K)], fbuf.at[p], semf.at[p])
        pltpu.async_copy(batch3.at[c], idx_v.at[p], semi.at[p])

    def wait_fetch(p):
        pltpu.make_async_copy(feat.at[pl.ds(0, K)], fbuf.at[p], semf.at[p]).wait()
        pltpu.make_async_copy(batch3.at[0], idx_v.at[p], semi.at[p]).wait()

    def issue_scatter(p):
        for h in range(2):
            pltpu.async_copy(
                sbuf.at[p, pl.ds(h * KH, KH)],
                acc.at[idx_s.at[p, h]],
                sems.at[p],
                add=True,
            )

    def wait_scatter(p):
        for h in range(2):
            pltpu.make_async_copy(
                sbuf.at[p, pl.ds(h * KH, KH)], acc.at[idx_s.at[p, h]], sems.at[p]
            ).wait()

    def process(c, t, p):
        wait_fetch(p)

        @pl.when(t >= 2)
        def _():
            wait_scatter(p)

        for v in range(K // L):
            idx_s[p, v // (KH // L), pl.ds((v % (KH // L)) * L, L)] = idx_v[
                p, v // (KH // L), pl.ds((v % (KH // L)) * L, L)
            ]

        fb = fbuf.at[p]
        sb = sbuf.at[p]

        def scale_row(r):
            fr = [fb[r, pl.ds(L * j, L)] for j in range(CJ)]
            m = [fr[j] * wregs[j] for j in range(CJ)]
            while len(m) > 1:
                m = [m[2 * i] + m[2 * i + 1] for i in range(len(m) // 2)]
            s = lane_sum_splat(m[0])
            t_ = bvec + s
            wgt = 1.0 / (1.0 + jnp.exp(-t_))
            for j in range(CJ):
                sb[r, pl.ds(L * j, L)] = fr[j] * wgt

        def row_body(rq, rc):
            for i in range(U):
                scale_row(rq * U + i)
            return rc

        lax.fori_loop(0, K // U, row_body, 0)
        issue_scatter(p)

        @pl.when(t + 2 < nch)
        def _():
            issue_fetch(c + 2, p)

    issue_fetch(start, 0)
    issue_fetch(start + 1, 1)

    def chunk_body(t, carry):
        c = start + t

        @pl.when(t % 2 == 0)
        def _():
            process(c, t, 0)

        @pl.when(t % 2 == 1)
        def _():
            process(c, t, 1)

        return carry

    lax.fori_loop(0, nch, chunk_body, 0)
    wait_scatter(0)
    wait_scatter(1)
    plsc.subcore_barrier()

    @pl.when(sid == 0)
    def _():
        pltpu.sync_copy(acc, out.at[cid])


@jax.jit
def _pooling(features, batch3, wflat, b16):
    mesh = plsc.VectorSubcoreMesh(core_axis_name="c", subcore_axis_name="s")
    kfn = functools.partial(
        pl.kernel,
        mesh=mesh,
        out_type=jax.ShapeDtypeStruct((NC, G, C), jnp.float32),
        scratch_types=[
            pltpu.VMEM_SHARED((G, C), jnp.float32),
            pltpu.VMEM((2, 2, KH), jnp.int32),
            pltpu.VMEM((2, 2, KH), jnp.int32),
            pltpu.VMEM((2, K, C), jnp.float32),
            pltpu.VMEM((2, K, C), jnp.float32),
            pltpu.VMEM((C,), jnp.float32),
            pltpu.VMEM((L,), jnp.float32),
            pltpu.VMEM((8, C), jnp.float32),
            pltpu.SemaphoreType.DMA((2,)),
            pltpu.SemaphoreType.DMA((2,)),
            pltpu.SemaphoreType.DMA((2,)),
        ],
    )(_body)
    return kfn(features, batch3, wflat, b16)


def kernel(features, batch, W, b):
    batch3 = batch.astype(jnp.int32).reshape(TCH, 2, KH)
    wflat = W.reshape(C).astype(jnp.float32)
    b16 = jnp.broadcast_to(b.reshape(()).astype(jnp.float32), (L,))
    partials = _pooling(features, batch3, wflat, b16)
    return partials[0] + partials[1]
